# baseline (speedup 1.0000x reference)
import jax
import jax.numpy as jnp
from jax.experimental import pallas as pl
from jax.experimental.pallas import tpu as pltpu

_EPS = 1e-5
_CHUNK = 2048


def _dot(a, b):
    return jax.lax.dot_general(a, b, (((1,), (0,)), ((), ())),
                               preferred_element_type=jnp.float32)


def _dot_t(a, b):
    return jax.lax.dot_general(a, b, (((1,), (1,)), ((), ())),
                               preferred_element_type=jnp.float32)


def _bn_act(y, g, b):
    m = jnp.mean(y, axis=1, keepdims=True)
    v = jnp.mean((y - m) ** 2, axis=1, keepdims=True)
    return jax.nn.relu(g * (y - m) / jnp.sqrt(v + _EPS) + b)


def _softmax_rows(y):
    z = y - jnp.max(y, axis=0, keepdims=True)
    e = jnp.exp(z)
    return e / jnp.sum(e, axis=0, keepdims=True)


def _top3_weights(d, iota):
    dw = d
    cnt = jnp.zeros_like(d)
    s = jnp.zeros(d.shape[:1] + (1,), d.dtype)
    for _ in range(3):
        m = jnp.min(dw, axis=1, keepdims=True)
        eq = dw == m
        ji = jnp.min(jnp.where(eq, iota, jnp.int32(1 << 30)), axis=1,
                     keepdims=True)
        E = iota == ji
        cnt = cnt + E.astype(jnp.float32)
        s = s + m
        dw = jnp.where(E, jnp.float32(3e38), dw)
    return (1.0 - d / s) * cnt


def _pair_d(a, b):
    d2 = ((a[0][:, None] - b[0][None, :]) ** 2
          + (a[1][:, None] - b[1][None, :]) ** 2
          + (a[2][:, None] - b[2][None, :]) ** 2)
    return jnp.sqrt(d2)


def _up_mat(hw_lo, w_hi):
    hw_hi = 4 * hw_lo
    ri = jax.lax.broadcasted_iota(jnp.int32, (hw_lo, hw_hi), 0)
    cj = jax.lax.broadcasted_iota(jnp.int32, (hw_lo, hw_hi), 1)
    parent = (cj // (2 * w_hi)) * (w_hi // 2) + (cj % w_hi) // 2
    return (ri == parent).astype(jnp.float32)


def _up2x_nat(t, u):
    c = t.shape[0]
    hw = t.shape[1] // 2
    tt = jnp.concatenate([t[:, :hw], t[:, hw:]], axis=0)
    up = _dot(tt, u)
    return jnp.concatenate([up[:c], up[c:]], axis=1)


def _point_kernel(nbf_ref, naf_ref, ig_ref, gf_ref, s32f_ref, s16f_ref,
                  na_ref, nb_ref,
                  baW0, baG0, baB0, baW1,
                  bpW0, bpG0, bpB0, bpW1, bpG1, bpB1, bpW2,
                  aaW0, aaG0, aaB0, aaW1,
                  apW0, apG0, apB0, apW1, apG1, apB1, apW2,
                  up_nb_out, up_na_out):
    nbf = nbf_ref[...]
    naf = naf_ref[...]
    ig = ig_ref[...]
    gf = gf_ref[...]
    x = jnp.concatenate([nbf, ig], axis=0)
    h = _bn_act(_dot(baW0[...], x), baG0[...], baB0[...])
    att = _softmax_rows(_dot(baW1[...], h))
    w32 = jnp.concatenate(
        [_dot(s32f_ref[b], att[:, b * 128:(b + 1) * 128]) for b in range(2)],
        axis=1)
    x2 = jnp.concatenate([nbf, gf, w32, ig], axis=0)
    h = _bn_act(_dot(bpW0[...], x2), bpG0[...], bpB0[...])
    h = _bn_act(_dot(bpW1[...], h), bpG1[...], bpB1[...])
    up_nb = _dot(bpW2[...], h)
    up_nb_out[...] = up_nb
    x3 = jnp.concatenate([naf, ig], axis=0)
    h = _bn_act(_dot(aaW0[...], x3), aaG0[...], aaB0[...])
    att16 = _softmax_rows(_dot(aaW1[...], h))
    w16 = jnp.concatenate(
        [_dot(s16f_ref[b], att16[:, b * 128:(b + 1) * 128]) for b in range(2)],
        axis=1)
    iota = jax.lax.broadcasted_iota(jnp.int32, (128, 128), 1)
    interp_ab = jnp.concatenate(
        [_dot_t(up_nb[:, b * 128:(b + 1) * 128],
                _top3_weights(_pair_d(na_ref[b], nb_ref[b]), iota))
         for b in range(2)], axis=1)
    x4 = jnp.concatenate([naf, interp_ab, w16], axis=0)
    h = _bn_act(_dot(apW0[...], x4), apG0[...], apB0[...])
    h = _bn_act(_dot(apW1[...], h), apG1[...], apB1[...])
    up_na_out[...] = _dot(apW2[...], h)


def _knn_kernel(pc_ref, na_ref, nb_ref, upnb_ref, upna_ref, idx_ref,
                pb_out, pa_out):
    pc = pc_ref[0]
    c = pc.shape[1]
    iota = jax.lax.broadcasted_iota(jnp.int32, (c, 128), 1)
    db = _pair_d(pc, nb_ref[0])
    wb = _top3_weights(db, iota)
    pb_out[0] = _dot_t(upnb_ref[0], wb)
    da = _pair_d(pc, na_ref[0])
    idx = idx_ref[0]
    cnt = ((idx[0][:, None] == iota).astype(jnp.float32)
           + (idx[1][:, None] == iota).astype(jnp.float32)
           + (idx[2][:, None] == iota).astype(jnp.float32))
    s = jnp.sum(cnt * da, axis=1, keepdims=True)
    wa = (1.0 - da / s) * cnt
    pa_out[0] = _dot_t(upna_ref[0], wa)


def _img_a(s32_ref, s16_ref, s8_ref, s4_ref, s2_ref, g32_ref, g16_ref,
           nbf_ref, naf_ref,
           a32W0, a32G0, a32B0, a32W1, a32G1, a32B1, a32W2,
           a16W0, a16G0, a16B0, a16W1, a16G1, a16B1, a16W2,
           u1W0, u1G0, u1B0, u1W1, u1G1, u1B1,
           u2W0, u2G0, u2B0, u2W1, u2G1, u2B1,
           u3W0,
           f8_out, r_out):
    s32 = s32_ref[...]
    s16 = s16_ref[...]
    x = jnp.concatenate([s32, g32_ref[...]], axis=0)
    h = _bn_act(_dot(a32W0[...], x), a32G0[...], a32B0[...])
    h = _bn_act(_dot(a32W1[...], h), a32G1[...], a32B1[...])
    a32 = _softmax_rows(_dot(a32W2[...], h))
    fus32 = jnp.concatenate(
        [jnp.concatenate(
            [_dot(nbf_ref[b], a32[:, b * 80:(b + 1) * 80]) for b in range(2)],
            axis=1), s32], axis=0)
    x = jnp.concatenate([s16, g16_ref[...]], axis=0)
    h = _bn_act(_dot(a16W0[...], x), a16G0[...], a16B0[...])
    h = _bn_act(_dot(a16W1[...], h), a16G1[...], a16B1[...])
    a16 = _softmax_rows(_dot(a16W2[...], h))
    fus16 = jnp.concatenate(
        [jnp.concatenate(
            [_dot(naf_ref[b], a16[:, b * 320:(b + 1) * 320]) for b in range(2)],
            axis=1), s16], axis=0)
    u1w = u1W0[...]
    y = (_up2x_nat(_dot(u1w[:, :768], fus32), _up_mat(80, 32))
         + _dot(u1w[:, 768:], fus16))
    h = _bn_act(y, u1G0[...], u1B0[...])
    f16 = _bn_act(_dot(u1W1[...], h), u1G1[...], u1B1[...])
    u2w = u2W0[...]
    y = (_up2x_nat(_dot(u2w[:, :256], f16), _up_mat(320, 64))
         + _dot(u2w[:, 256:], s8_ref[...]))
    h = _bn_act(y, u2G0[...], u2B0[...])
    f8_out[...] = _bn_act(_dot(u2W1[...], h), u2G1[...], u2B1[...])
    u3w = u3W0[...]
    hsel = (2 * jax.lax.broadcasted_iota(jnp.int32, (40, 80), 0)
            == jax.lax.broadcasted_iota(jnp.int32, (40, 80), 1)
            ).astype(jnp.float32)
    wsel = (jax.lax.broadcasted_iota(jnp.int32, (256, 128), 0)
            == 2 * jax.lax.broadcasted_iota(jnp.int32, (256, 128), 1)
            ).astype(jnp.float32)
    r2 = []
    for b in range(2):
        rh = jax.lax.dot_general(hsel, s2_ref[b],
                                 (((1,), (1,)), ((), ())),
                                 preferred_element_type=jnp.float32)
        rh = jnp.transpose(rh, (1, 0, 2))
        qw = jax.lax.dot_general(rh, wsel, (((2,), (0,)), ((), ())),
                                 preferred_element_type=jnp.float32)
        r2.append(qw.reshape(64, 5120))
    r_out[...] = (_dot(u3w[:, 128:192], s4_ref[...])
                  + _dot(u3w[:, 192:], jnp.concatenate(r2, axis=1)))


def _img_b(f8_ref, r_ref,
           u3W0, u3G0, u3B0, u3W1, u3G1, u3B1,
           out_ref, y_scr, z_scr, m2_scr, v2_scr):
    s = pl.program_id(0)
    u3w = u3W0[...]

    for k in range(4):
        @pl.when(s == k)
        def _build(k=k):
            dh3, dw3 = k // 2, k % 2
            t = _dot(u3w[:, :128], f8_ref[...])
            r = r_ref[...]
            wsel = (jax.lax.broadcasted_iota(jnp.int32, (128, 64), 0)
                    == 2 * jax.lax.broadcasted_iota(jnp.int32, (128, 64), 1)
                    + dw3).astype(jnp.float32)
            hsel = (2 * jax.lax.broadcasted_iota(jnp.int32, (20, 40), 0)
                    + dh3
                    == jax.lax.broadcasted_iota(jnp.int32, (20, 40), 1)
                    ).astype(jnp.float32)
            parts = []
            for b in range(2):
                rb = r[:, b * 5120:(b + 1) * 5120].reshape(64, 40, 128)
                rh = jax.lax.dot_general(
                    hsel, rb, (((1,), (1,)), ((), ())),
                    preferred_element_type=jnp.float32)
                rh = jnp.transpose(rh, (1, 0, 2))
                rv = jax.lax.dot_general(
                    rh, wsel, (((2,), (0,)), ((), ())),
                    preferred_element_type=jnp.float32)
                parts.append(rv.reshape(64, 1280))
            y_scr[k] = t + jnp.concatenate(parts, axis=1)

    @pl.when(s == 4)
    def _stats():
        y_all = y_scr[...]
        m = jnp.mean(y_all, axis=(0, 2), keepdims=True)
        v = jnp.mean((y_all - m) ** 2, axis=(0, 2), keepdims=True)
        g0 = u3G0[...]
        b0 = u3B0[...]
        for q in range(4):
            hq = jax.nn.relu(g0 * (y_scr[q] - m[0]) / jnp.sqrt(v[0] + _EPS)
                             + b0)
            z_scr[q] = _dot(u3W1[...], hq)
        z_all = z_scr[...]
        m2 = jnp.mean(z_all, axis=(0, 2), keepdims=True)
        v2 = jnp.mean((z_all - m2) ** 2, axis=(0, 2), keepdims=True)
        m2_scr[...] = m2[0]
        v2_scr[...] = v2[0]

    for k in range(4, 8):
        @pl.when(s == k)
        def _write(k=k):
            z = z_scr[k - 4]
            o = jax.nn.relu(u3G1[...] * (z - m2_scr[...])
                            / jnp.sqrt(v2_scr[...] + _EPS) + u3B1[...])
            o4 = jnp.concatenate([o[:, :1280].reshape(1, 64, 20, 64),
                                  o[:, 1280:].reshape(1, 64, 20, 64)],
                                 axis=0)
            out_ref[...] = o4.reshape(1, 1, 2, 64, 20, 64)


def _cols(x):
    return x.transpose(1, 0, 2).reshape(x.shape[1], -1)


def _layers(p):
    out = []
    for w, g, b in p:
        out.extend([w, g.reshape(-1, 1), b.reshape(-1, 1)])
    return out


def _layers_nolast(p):
    out = []
    for w, g, b in p[:-1]:
        out.extend([w, g.reshape(-1, 1), b.reshape(-1, 1)])
    out.append(p[-1][0])
    return out


def kernel(pc, node_a, node_b, img_s32_feature_map, img_s16_feature_map,
           img_s8_feature_map, img_s4_feature_map, img_s2_feature_map,
           img_global_feature, global_feature, node_b_features,
           node_a_features, node_a_min_k_idx, params):
    f32 = jnp.float32
    n = pc.shape[2]
    nbf_c = _cols(node_b_features)
    naf_c = _cols(node_a_features)
    ig_c = jnp.broadcast_to(img_global_feature.T[:, :, None],
                            (512, 2, 128)).reshape(512, 256)
    gf_c = jnp.broadcast_to(global_feature.transpose(1, 0, 2),
                            (512, 2, 128)).reshape(512, 256)
    s32f = img_s32_feature_map.reshape(2, 512, 80)
    s16f = img_s16_feature_map.reshape(2, 256, 320)
    p = params
    up_nb_c, up_na_c = pl.pallas_call(
        _point_kernel,
        out_shape=[jax.ShapeDtypeStruct((512, 256), f32),
                   jax.ShapeDtypeStruct((128, 256), f32)],
    )(nbf_c, naf_c, ig_c, gf_c, s32f, s16f, node_a, node_b,
      *_layers_nolast(p["nb_att"]), *_layers_nolast(p["nb_pn"]),
      *_layers_nolast(p["na_att"]), *_layers_nolast(p["na_pn"]))

    up_nb3 = up_nb_c.reshape(512, 2, 128).transpose(1, 0, 2)
    up_na3 = up_na_c.reshape(128, 2, 128).transpose(1, 0, 2)
    idx_t = node_a_min_k_idx.astype(jnp.int32).transpose(0, 2, 1)
    interp_pb, interp_pa = pl.pallas_call(
        _knn_kernel,
        grid=(2, n // _CHUNK),
        compiler_params=pltpu.CompilerParams(
            dimension_semantics=("parallel", "parallel")),
        in_specs=[
            pl.BlockSpec((1, 3, _CHUNK), lambda b, i: (b, 0, i)),
            pl.BlockSpec((1, 3, 128), lambda b, i: (b, 0, 0)),
            pl.BlockSpec((1, 3, 128), lambda b, i: (b, 0, 0)),
            pl.BlockSpec((1, 512, 128), lambda b, i: (b, 0, 0)),
            pl.BlockSpec((1, 128, 128), lambda b, i: (b, 0, 0)),
            pl.BlockSpec((1, 3, _CHUNK), lambda b, i: (b, 0, i)),
        ],
        out_specs=[
            pl.BlockSpec((1, 512, _CHUNK), lambda b, i: (b, 0, i)),
            pl.BlockSpec((1, 128, _CHUNK), lambda b, i: (b, 0, i)),
        ],
        out_shape=[jax.ShapeDtypeStruct((2, 512, n), f32),
                   jax.ShapeDtypeStruct((2, 128, n), f32)],
    )(pc, node_a, node_b, up_nb3, up_na3, idx_t)

    s32c = _cols(s32f)
    s16c = _cols(img_s16_feature_map.reshape(2, 256, 320))
    s8c = _cols(img_s8_feature_map.reshape(2, 128, 1280))
    g32c = jnp.broadcast_to(global_feature.transpose(1, 0, 2),
                            (512, 2, 80)).reshape(512, 160)
    g16c = jnp.broadcast_to(global_feature.transpose(1, 0, 2),
                            (512, 2, 320)).reshape(512, 640)
    s4c = _cols(img_s4_feature_map.reshape(2, 64, 5120))
    f8c, rc = pl.pallas_call(
        _img_a,
        out_shape=[jax.ShapeDtypeStruct((128, 2560), f32),
                   jax.ShapeDtypeStruct((64, 10240), f32)],
    )(s32c, s16c, s8c, s4c, img_s2_feature_map, g32c, g16c,
      node_b_features, node_a_features,
      *_layers_nolast(p["att32"]), *_layers_nolast(p["att16"]),
      *_layers(p["up1"]), *_layers(p["up2"]), p["up3"][0][0])

    up3l = _layers(p["up3"])
    fmid_r = pl.pallas_call(
        _img_b,
        grid=(8,),
        in_specs=[
            pl.BlockSpec((128, 2560), lambda s: (0, 0)),
            pl.BlockSpec((64, 10240), lambda s: (0, 0)),
        ] + [pl.BlockSpec(w.shape, lambda s: (0, 0)) for w in up3l],
        out_specs=pl.BlockSpec(
            (1, 1, 2, 64, 20, 64),
            lambda s: (jnp.maximum(s - 4, 0) // 2, jnp.maximum(s - 4, 0) % 2,
                       0, 0, 0, 0)),
        out_shape=jax.ShapeDtypeStruct((2, 2, 2, 64, 20, 64), f32),
        scratch_shapes=[pltpu.VMEM((4, 64, 2560), f32),
                        pltpu.VMEM((4, 64, 2560), f32),
                        pltpu.VMEM((64, 1), f32),
                        pltpu.VMEM((64, 1), f32)],
    )(f8c, rc, *up3l)
    fmid = (fmid_r.transpose(2, 3, 4, 0, 5, 1).reshape(2, 64, 40, 128))
    return (fmid, interp_pa, interp_pb)


# speedup vs baseline: 1.0117x; 1.0117x over previous
import jax
import jax.numpy as jnp
from jax.experimental import pallas as pl
from jax.experimental.pallas import tpu as pltpu

_EPS = 1e-5
_CHUNK = 4096


def _dot(a, b):
    return jax.lax.dot_general(a, b, (((1,), (0,)), ((), ())),
                               preferred_element_type=jnp.float32)


def _dot_t(a, b):
    return jax.lax.dot_general(a, b, (((1,), (1,)), ((), ())),
                               preferred_element_type=jnp.float32)


def _bn_act(y, g, b):
    m = jnp.mean(y, axis=1, keepdims=True)
    v = jnp.mean((y - m) ** 2, axis=1, keepdims=True)
    return jax.nn.relu(g * (y - m) / jnp.sqrt(v + _EPS) + b)


def _softmax_rows(y):
    z = y - jnp.max(y, axis=0, keepdims=True)
    e = jnp.exp(z)
    return e / jnp.sum(e, axis=0, keepdims=True)


def _top3_weights(d, iota):
    dw = d
    cnt = jnp.zeros_like(d)
    s = jnp.zeros(d.shape[:1] + (1,), d.dtype)
    for _ in range(3):
        m = jnp.min(dw, axis=1, keepdims=True)
        eq = dw == m
        ji = jnp.min(jnp.where(eq, iota, jnp.int32(1 << 30)), axis=1,
                     keepdims=True)
        E = iota == ji
        cnt = cnt + E.astype(jnp.float32)
        s = s + m
        dw = jnp.where(E, jnp.float32(3e38), dw)
    return (1.0 - d / s) * cnt


def _pair_d(a, b):
    d2 = ((a[0][:, None] - b[0][None, :]) ** 2
          + (a[1][:, None] - b[1][None, :]) ** 2
          + (a[2][:, None] - b[2][None, :]) ** 2)
    return jnp.sqrt(d2)


def _up_mat(hw_lo, w_hi):
    hw_hi = 4 * hw_lo
    ri = jax.lax.broadcasted_iota(jnp.int32, (hw_lo, hw_hi), 0)
    cj = jax.lax.broadcasted_iota(jnp.int32, (hw_lo, hw_hi), 1)
    parent = (cj // (2 * w_hi)) * (w_hi // 2) + (cj % w_hi) // 2
    return (ri == parent).astype(jnp.float32)


def _up2x_nat(t, u):
    c = t.shape[0]
    hw = t.shape[1] // 2
    tt = jnp.concatenate([t[:, :hw], t[:, hw:]], axis=0)
    up = _dot(tt, u)
    return jnp.concatenate([up[:c], up[c:]], axis=1)


def _point_kernel(nbf_ref, naf_ref, ig_ref, gf_ref, s32f_ref, s16f_ref,
                  na_ref, nb_ref,
                  baW0, baG0, baB0, baW1,
                  bpW0, bpG0, bpB0, bpW1, bpG1, bpB1, bpW2,
                  aaW0, aaG0, aaB0, aaW1,
                  apW0, apG0, apB0, apW1, apG1, apB1, apW2,
                  up_nb_out, up_na_out):
    nbf = nbf_ref[...]
    naf = naf_ref[...]
    ig = ig_ref[...]
    gf = gf_ref[...]
    x = jnp.concatenate([nbf, ig], axis=0)
    h = _bn_act(_dot(baW0[...], x), baG0[...], baB0[...])
    att = _softmax_rows(_dot(baW1[...], h))
    w32 = jnp.concatenate(
        [_dot(s32f_ref[b], att[:, b * 128:(b + 1) * 128]) for b in range(2)],
        axis=1)
    x2 = jnp.concatenate([nbf, gf, w32, ig], axis=0)
    h = _bn_act(_dot(bpW0[...], x2), bpG0[...], bpB0[...])
    h = _bn_act(_dot(bpW1[...], h), bpG1[...], bpB1[...])
    up_nb = _dot(bpW2[...], h)
    up_nb_out[...] = up_nb
    x3 = jnp.concatenate([naf, ig], axis=0)
    h = _bn_act(_dot(aaW0[...], x3), aaG0[...], aaB0[...])
    att16 = _softmax_rows(_dot(aaW1[...], h))
    w16 = jnp.concatenate(
        [_dot(s16f_ref[b], att16[:, b * 128:(b + 1) * 128]) for b in range(2)],
        axis=1)
    iota = jax.lax.broadcasted_iota(jnp.int32, (128, 128), 1)
    interp_ab = jnp.concatenate(
        [_dot_t(up_nb[:, b * 128:(b + 1) * 128],
                _top3_weights(_pair_d(na_ref[b], nb_ref[b]), iota))
         for b in range(2)], axis=1)
    x4 = jnp.concatenate([naf, interp_ab, w16], axis=0)
    h = _bn_act(_dot(apW0[...], x4), apG0[...], apB0[...])
    h = _bn_act(_dot(apW1[...], h), apG1[...], apB1[...])
    up_na_out[...] = _dot(apW2[...], h)


def _knn_kernel(pc_ref, na_ref, nb_ref, upnb_ref, upna_ref, idx_ref,
                pb_out, pa_out):
    pc = pc_ref[0]
    c = pc.shape[1]
    iota = jax.lax.broadcasted_iota(jnp.int32, (c, 128), 1)
    db = _pair_d(pc, nb_ref[0])
    wb = _top3_weights(db, iota)
    pb_out[0] = _dot_t(upnb_ref[0], wb)
    da = _pair_d(pc, na_ref[0])
    idx = idx_ref[0]
    cnt = ((idx[0][:, None] == iota).astype(jnp.float32)
           + (idx[1][:, None] == iota).astype(jnp.float32)
           + (idx[2][:, None] == iota).astype(jnp.float32))
    s = jnp.sum(cnt * da, axis=1, keepdims=True)
    wa = (1.0 - da / s) * cnt
    pa_out[0] = _dot_t(upna_ref[0], wa)


def _img_a(s32_ref, s16_ref, s8_ref, s4_ref, s2_ref, g32_ref, g16_ref,
           nbf_ref, naf_ref,
           a32W0, a32G0, a32B0, a32W1, a32G1, a32B1, a32W2,
           a16W0, a16G0, a16B0, a16W1, a16G1, a16B1, a16W2,
           u1W0, u1G0, u1B0, u1W1, u1G1, u1B1,
           u2W0, u2G0, u2B0, u2W1, u2G1, u2B1,
           u3W0,
           f8_out, r_out):
    s32 = s32_ref[...]
    s16 = s16_ref[...]
    x = jnp.concatenate([s32, g32_ref[...]], axis=0)
    h = _bn_act(_dot(a32W0[...], x), a32G0[...], a32B0[...])
    h = _bn_act(_dot(a32W1[...], h), a32G1[...], a32B1[...])
    a32 = _softmax_rows(_dot(a32W2[...], h))
    fus32 = jnp.concatenate(
        [jnp.concatenate(
            [_dot(nbf_ref[b], a32[:, b * 80:(b + 1) * 80]) for b in range(2)],
            axis=1), s32], axis=0)
    x = jnp.concatenate([s16, g16_ref[...]], axis=0)
    h = _bn_act(_dot(a16W0[...], x), a16G0[...], a16B0[...])
    h = _bn_act(_dot(a16W1[...], h), a16G1[...], a16B1[...])
    a16 = _softmax_rows(_dot(a16W2[...], h))
    fus16 = jnp.concatenate(
        [jnp.concatenate(
            [_dot(naf_ref[b], a16[:, b * 320:(b + 1) * 320]) for b in range(2)],
            axis=1), s16], axis=0)
    u1w = u1W0[...]
    y = (_up2x_nat(_dot(u1w[:, :768], fus32), _up_mat(80, 32))
         + _dot(u1w[:, 768:], fus16))
    h = _bn_act(y, u1G0[...], u1B0[...])
    f16 = _bn_act(_dot(u1W1[...], h), u1G1[...], u1B1[...])
    u2w = u2W0[...]
    y = (_up2x_nat(_dot(u2w[:, :256], f16), _up_mat(320, 64))
         + _dot(u2w[:, 256:], s8_ref[...]))
    h = _bn_act(y, u2G0[...], u2B0[...])
    f8_out[...] = _bn_act(_dot(u2W1[...], h), u2G1[...], u2B1[...])
    u3w = u3W0[...]
    hsel = (2 * jax.lax.broadcasted_iota(jnp.int32, (40, 80), 0)
            == jax.lax.broadcasted_iota(jnp.int32, (40, 80), 1)
            ).astype(jnp.float32)
    wsel = (jax.lax.broadcasted_iota(jnp.int32, (256, 128), 0)
            == 2 * jax.lax.broadcasted_iota(jnp.int32, (256, 128), 1)
            ).astype(jnp.float32)
    r2 = []
    for b in range(2):
        rh = jax.lax.dot_general(hsel, s2_ref[b],
                                 (((1,), (1,)), ((), ())),
                                 preferred_element_type=jnp.float32)
        rh = jnp.transpose(rh, (1, 0, 2))
        qw = jax.lax.dot_general(rh, wsel, (((2,), (0,)), ((), ())),
                                 preferred_element_type=jnp.float32)
        r2.append(qw.reshape(64, 5120))
    r_out[...] = (_dot(u3w[:, 128:192], s4_ref[...])
                  + _dot(u3w[:, 192:], jnp.concatenate(r2, axis=1)))


def _img_b(f8_ref, r_ref,
           u3W0, u3G0, u3B0, u3W1, u3G1, u3B1,
           out_ref, y_scr, z_scr, m2_scr, v2_scr):
    s = pl.program_id(0)
    u3w = u3W0[...]

    for k in range(4):
        @pl.when(s == k)
        def _build(k=k):
            dh3, dw3 = k // 2, k % 2
            t = _dot(u3w[:, :128], f8_ref[...])
            r = r_ref[...]
            wsel = (jax.lax.broadcasted_iota(jnp.int32, (128, 64), 0)
                    == 2 * jax.lax.broadcasted_iota(jnp.int32, (128, 64), 1)
                    + dw3).astype(jnp.float32)
            hsel = (2 * jax.lax.broadcasted_iota(jnp.int32, (20, 40), 0)
                    + dh3
                    == jax.lax.broadcasted_iota(jnp.int32, (20, 40), 1)
                    ).astype(jnp.float32)
            parts = []
            for b in range(2):
                rb = r[:, b * 5120:(b + 1) * 5120].reshape(64, 40, 128)
                rh = jax.lax.dot_general(
                    hsel, rb, (((1,), (1,)), ((), ())),
                    preferred_element_type=jnp.float32)
                rh = jnp.transpose(rh, (1, 0, 2))
                rv = jax.lax.dot_general(
                    rh, wsel, (((2,), (0,)), ((), ())),
                    preferred_element_type=jnp.float32)
                parts.append(rv.reshape(64, 1280))
            y_scr[k] = t + jnp.concatenate(parts, axis=1)

    @pl.when(s == 4)
    def _stats():
        y_all = y_scr[...]
        m = jnp.mean(y_all, axis=(0, 2), keepdims=True)
        v = jnp.mean((y_all - m) ** 2, axis=(0, 2), keepdims=True)
        g0 = u3G0[...]
        b0 = u3B0[...]
        for q in range(4):
            hq = jax.nn.relu(g0 * (y_scr[q] - m[0]) / jnp.sqrt(v[0] + _EPS)
                             + b0)
            z_scr[q] = _dot(u3W1[...], hq)
        z_all = z_scr[...]
        m2 = jnp.mean(z_all, axis=(0, 2), keepdims=True)
        v2 = jnp.mean((z_all - m2) ** 2, axis=(0, 2), keepdims=True)
        m2_scr[...] = m2[0]
        v2_scr[...] = v2[0]

    for k in range(4, 8):
        @pl.when(s == k)
        def _write(k=k):
            z = z_scr[k - 4]
            o = jax.nn.relu(u3G1[...] * (z - m2_scr[...])
                            / jnp.sqrt(v2_scr[...] + _EPS) + u3B1[...])
            o4 = jnp.concatenate([o[:, :1280].reshape(1, 64, 20, 64),
                                  o[:, 1280:].reshape(1, 64, 20, 64)],
                                 axis=0)
            out_ref[...] = o4.reshape(1, 1, 2, 64, 20, 64)


def _cols(x):
    return x.transpose(1, 0, 2).reshape(x.shape[1], -1)


def _layers(p):
    out = []
    for w, g, b in p:
        out.extend([w, g.reshape(-1, 1), b.reshape(-1, 1)])
    return out


def _layers_nolast(p):
    out = []
    for w, g, b in p[:-1]:
        out.extend([w, g.reshape(-1, 1), b.reshape(-1, 1)])
    out.append(p[-1][0])
    return out


def kernel(pc, node_a, node_b, img_s32_feature_map, img_s16_feature_map,
           img_s8_feature_map, img_s4_feature_map, img_s2_feature_map,
           img_global_feature, global_feature, node_b_features,
           node_a_features, node_a_min_k_idx, params):
    f32 = jnp.float32
    n = pc.shape[2]
    nbf_c = _cols(node_b_features)
    naf_c = _cols(node_a_features)
    ig_c = jnp.broadcast_to(img_global_feature.T[:, :, None],
                            (512, 2, 128)).reshape(512, 256)
    gf_c = jnp.broadcast_to(global_feature.transpose(1, 0, 2),
                            (512, 2, 128)).reshape(512, 256)
    s32f = img_s32_feature_map.reshape(2, 512, 80)
    s16f = img_s16_feature_map.reshape(2, 256, 320)
    p = params
    up_nb_c, up_na_c = pl.pallas_call(
        _point_kernel,
        out_shape=[jax.ShapeDtypeStruct((512, 256), f32),
                   jax.ShapeDtypeStruct((128, 256), f32)],
    )(nbf_c, naf_c, ig_c, gf_c, s32f, s16f, node_a, node_b,
      *_layers_nolast(p["nb_att"]), *_layers_nolast(p["nb_pn"]),
      *_layers_nolast(p["na_att"]), *_layers_nolast(p["na_pn"]))

    up_nb3 = up_nb_c.reshape(512, 2, 128).transpose(1, 0, 2)
    up_na3 = up_na_c.reshape(128, 2, 128).transpose(1, 0, 2)
    idx_t = node_a_min_k_idx.astype(jnp.int32).transpose(0, 2, 1)
    interp_pb, interp_pa = pl.pallas_call(
        _knn_kernel,
        grid=(2, n // _CHUNK),
        compiler_params=pltpu.CompilerParams(
            dimension_semantics=("parallel", "parallel")),
        in_specs=[
            pl.BlockSpec((1, 3, _CHUNK), lambda b, i: (b, 0, i)),
            pl.BlockSpec((1, 3, 128), lambda b, i: (b, 0, 0)),
            pl.BlockSpec((1, 3, 128), lambda b, i: (b, 0, 0)),
            pl.BlockSpec((1, 512, 128), lambda b, i: (b, 0, 0)),
            pl.BlockSpec((1, 128, 128), lambda b, i: (b, 0, 0)),
            pl.BlockSpec((1, 3, _CHUNK), lambda b, i: (b, 0, i)),
        ],
        out_specs=[
            pl.BlockSpec((1, 512, _CHUNK), lambda b, i: (b, 0, i)),
            pl.BlockSpec((1, 128, _CHUNK), lambda b, i: (b, 0, i)),
        ],
        out_shape=[jax.ShapeDtypeStruct((2, 512, n), f32),
                   jax.ShapeDtypeStruct((2, 128, n), f32)],
    )(pc, node_a, node_b, up_nb3, up_na3, idx_t)

    s32c = _cols(s32f)
    s16c = _cols(img_s16_feature_map.reshape(2, 256, 320))
    s8c = _cols(img_s8_feature_map.reshape(2, 128, 1280))
    g32c = jnp.broadcast_to(global_feature.transpose(1, 0, 2),
                            (512, 2, 80)).reshape(512, 160)
    g16c = jnp.broadcast_to(global_feature.transpose(1, 0, 2),
                            (512, 2, 320)).reshape(512, 640)
    s4c = _cols(img_s4_feature_map.reshape(2, 64, 5120))
    f8c, rc = pl.pallas_call(
        _img_a,
        out_shape=[jax.ShapeDtypeStruct((128, 2560), f32),
                   jax.ShapeDtypeStruct((64, 10240), f32)],
    )(s32c, s16c, s8c, s4c, img_s2_feature_map, g32c, g16c,
      node_b_features, node_a_features,
      *_layers_nolast(p["att32"]), *_layers_nolast(p["att16"]),
      *_layers(p["up1"]), *_layers(p["up2"]), p["up3"][0][0])

    up3l = _layers(p["up3"])
    fmid_r = pl.pallas_call(
        _img_b,
        grid=(8,),
        in_specs=[
            pl.BlockSpec((128, 2560), lambda s: (0, 0)),
            pl.BlockSpec((64, 10240), lambda s: (0, 0)),
        ] + [pl.BlockSpec(w.shape, lambda s: (0, 0)) for w in up3l],
        out_specs=pl.BlockSpec(
            (1, 1, 2, 64, 20, 64),
            lambda s: (jnp.maximum(s - 4, 0) // 2, jnp.maximum(s - 4, 0) % 2,
                       0, 0, 0, 0)),
        out_shape=jax.ShapeDtypeStruct((2, 2, 2, 64, 20, 64), f32),
        scratch_shapes=[pltpu.VMEM((4, 64, 2560), f32),
                        pltpu.VMEM((4, 64, 2560), f32),
                        pltpu.VMEM((64, 1), f32),
                        pltpu.VMEM((64, 1), f32)],
    )(f8c, rc, *up3l)
    fmid = (fmid_r.transpose(2, 3, 4, 0, 5, 1).reshape(2, 64, 40, 128))
    return (fmid, interp_pa, interp_pb)


# speedup vs baseline: 1.0744x; 1.0619x over previous
import jax
import jax.numpy as jnp
from jax.experimental import pallas as pl
from jax.experimental.pallas import tpu as pltpu

_EPS = 1e-5
_CHUNK = 4096


def _dot(a, b):
    return jax.lax.dot_general(a, b, (((1,), (0,)), ((), ())),
                               preferred_element_type=jnp.float32)


def _dot_t(a, b):
    return jax.lax.dot_general(a, b, (((1,), (1,)), ((), ())),
                               preferred_element_type=jnp.float32)


def _bn_act(y, g, b):
    m = jnp.mean(y, axis=1, keepdims=True)
    v = jnp.mean((y - m) ** 2, axis=1, keepdims=True)
    return jax.nn.relu(g * (y - m) / jnp.sqrt(v + _EPS) + b)


def _softmax_rows(y):
    z = y - jnp.max(y, axis=0, keepdims=True)
    e = jnp.exp(z)
    return e / jnp.sum(e, axis=0, keepdims=True)


def _top3_weights(d, iota):
    dw = d
    cnt = jnp.zeros_like(d)
    s = jnp.zeros(d.shape[:1] + (1,), d.dtype)
    for _ in range(3):
        m = jnp.min(dw, axis=1, keepdims=True)
        eq = dw == m
        ji = jnp.min(jnp.where(eq, iota, jnp.int32(1 << 30)), axis=1,
                     keepdims=True)
        E = iota == ji
        cnt = cnt + E.astype(jnp.float32)
        s = s + m
        dw = jnp.where(E, jnp.float32(3e38), dw)
    return (1.0 - d / s) * cnt


def _pair_d(a, b):
    d2 = ((a[0][:, None] - b[0][None, :]) ** 2
          + (a[1][:, None] - b[1][None, :]) ** 2
          + (a[2][:, None] - b[2][None, :]) ** 2)
    return jnp.sqrt(d2)


def _up_mat(hw_lo, w_hi):
    hw_hi = 4 * hw_lo
    ri = jax.lax.broadcasted_iota(jnp.int32, (hw_lo, hw_hi), 0)
    cj = jax.lax.broadcasted_iota(jnp.int32, (hw_lo, hw_hi), 1)
    parent = (cj // (2 * w_hi)) * (w_hi // 2) + (cj % w_hi) // 2
    return (ri == parent).astype(jnp.float32)


def _up2x_nat(t, u):
    c = t.shape[0]
    hw = t.shape[1] // 2
    tt = jnp.concatenate([t[:, :hw], t[:, hw:]], axis=0)
    up = _dot(tt, u)
    return jnp.concatenate([up[:c], up[c:]], axis=1)


def _point_kernel(nbf_ref, naf_ref, ig_ref, gf_ref, s32f_ref, s16f_ref,
                  na_ref, nb_ref,
                  baW0, baG0, baB0, baW1,
                  bpW0, bpG0, bpB0, bpW1, bpG1, bpB1, bpW2,
                  aaW0, aaG0, aaB0, aaW1,
                  apW0, apG0, apB0, apW1, apG1, apB1, apW2,
                  up_nb_out, up_na_out):
    nbf = nbf_ref[...]
    naf = naf_ref[...]
    ig = ig_ref[...]
    gf = gf_ref[...]
    x = jnp.concatenate([nbf, ig], axis=0)
    h = _bn_act(_dot(baW0[...], x), baG0[...], baB0[...])
    att = _softmax_rows(_dot(baW1[...], h))
    w32 = jnp.concatenate(
        [_dot(s32f_ref[b], att[:, b * 128:(b + 1) * 128]) for b in range(2)],
        axis=1)
    x2 = jnp.concatenate([nbf, gf, w32, ig], axis=0)
    h = _bn_act(_dot(bpW0[...], x2), bpG0[...], bpB0[...])
    h = _bn_act(_dot(bpW1[...], h), bpG1[...], bpB1[...])
    up_nb = _dot(bpW2[...], h)
    up_nb_out[...] = up_nb
    x3 = jnp.concatenate([naf, ig], axis=0)
    h = _bn_act(_dot(aaW0[...], x3), aaG0[...], aaB0[...])
    att16 = _softmax_rows(_dot(aaW1[...], h))
    w16 = jnp.concatenate(
        [_dot(s16f_ref[b], att16[:, b * 128:(b + 1) * 128]) for b in range(2)],
        axis=1)
    iota = jax.lax.broadcasted_iota(jnp.int32, (128, 128), 1)
    interp_ab = jnp.concatenate(
        [_dot_t(up_nb[:, b * 128:(b + 1) * 128],
                _top3_weights(_pair_d(na_ref[b], nb_ref[b]), iota))
         for b in range(2)], axis=1)
    x4 = jnp.concatenate([naf, interp_ab, w16], axis=0)
    h = _bn_act(_dot(apW0[...], x4), apG0[...], apB0[...])
    h = _bn_act(_dot(apW1[...], h), apG1[...], apB1[...])
    up_na_out[...] = _dot(apW2[...], h)


def _knn_kernel(pc_ref, na_ref, nb_ref, upnb_ref, upna_ref, idx_ref,
                pb_out, pa_out):
    pc = pc_ref[0]
    c = pc.shape[1]
    iota = jax.lax.broadcasted_iota(jnp.int32, (c, 128), 1)
    db = _pair_d(pc, nb_ref[0])
    wb = _top3_weights(db, iota)
    pb_out[0] = _dot_t(upnb_ref[0], wb)
    da = _pair_d(pc, na_ref[0])
    idx = idx_ref[0]
    cnt = ((idx[0][:, None] == iota).astype(jnp.float32)
           + (idx[1][:, None] == iota).astype(jnp.float32)
           + (idx[2][:, None] == iota).astype(jnp.float32))
    s = jnp.sum(cnt * da, axis=1, keepdims=True)
    wa = (1.0 - da / s) * cnt
    pa_out[0] = _dot_t(upna_ref[0], wa)


def _img_a(s32_ref, s16_ref, s8_ref, s4_ref, s2_ref, g32_ref, g16_ref,
           nbf_ref, naf_ref,
           a32W0, a32G0, a32B0, a32W1, a32G1, a32B1, a32W2,
           a16W0, a16G0, a16B0, a16W1, a16G1, a16B1, a16W2,
           u1W0, u1G0, u1B0, u1W1, u1G1, u1B1,
           u2W0, u2G0, u2B0, u2W1, u2G1, u2B1,
           u3W0,
           f8_out, r_out):
    s32 = s32_ref[...]
    s16 = s16_ref[...]
    x = jnp.concatenate([s32, g32_ref[...]], axis=0)
    h = _bn_act(_dot(a32W0[...], x), a32G0[...], a32B0[...])
    h = _bn_act(_dot(a32W1[...], h), a32G1[...], a32B1[...])
    a32 = _softmax_rows(_dot(a32W2[...], h))
    fus32 = jnp.concatenate(
        [jnp.concatenate(
            [_dot(nbf_ref[b], a32[:, b * 80:(b + 1) * 80]) for b in range(2)],
            axis=1), s32], axis=0)
    x = jnp.concatenate([s16, g16_ref[...]], axis=0)
    h = _bn_act(_dot(a16W0[...], x), a16G0[...], a16B0[...])
    h = _bn_act(_dot(a16W1[...], h), a16G1[...], a16B1[...])
    a16 = _softmax_rows(_dot(a16W2[...], h))
    fus16 = jnp.concatenate(
        [jnp.concatenate(
            [_dot(naf_ref[b], a16[:, b * 320:(b + 1) * 320]) for b in range(2)],
            axis=1), s16], axis=0)
    u1w = u1W0[...]
    y = (_up2x_nat(_dot(u1w[:, :768], fus32), _up_mat(80, 32))
         + _dot(u1w[:, 768:], fus16))
    h = _bn_act(y, u1G0[...], u1B0[...])
    f16 = _bn_act(_dot(u1W1[...], h), u1G1[...], u1B1[...])
    u2w = u2W0[...]
    y = (_up2x_nat(_dot(u2w[:, :256], f16), _up_mat(320, 64))
         + _dot(u2w[:, 256:], s8_ref[...]))
    h = _bn_act(y, u2G0[...], u2B0[...])
    f8_out[...] = _bn_act(_dot(u2W1[...], h), u2G1[...], u2B1[...])
    u3w = u3W0[...]
    hsel = (2 * jax.lax.broadcasted_iota(jnp.int32, (40, 80), 0)
            == jax.lax.broadcasted_iota(jnp.int32, (40, 80), 1)
            ).astype(jnp.float32)
    wsel = (jax.lax.broadcasted_iota(jnp.int32, (256, 128), 0)
            == 2 * jax.lax.broadcasted_iota(jnp.int32, (256, 128), 1)
            ).astype(jnp.float32)
    r2 = []
    for b in range(2):
        rh = jax.lax.dot_general(hsel, s2_ref[b],
                                 (((1,), (1,)), ((), ())),
                                 preferred_element_type=jnp.float32)
        rh = jnp.transpose(rh, (1, 0, 2))
        qw = jax.lax.dot_general(rh, wsel, (((2,), (0,)), ((), ())),
                                 preferred_element_type=jnp.float32)
        r2.append(qw.reshape(64, 5120))
    s4c = jnp.concatenate([s4_ref[0].reshape(64, 5120),
                           s4_ref[1].reshape(64, 5120)], axis=1)
    r_out[...] = (_dot(u3w[:, 128:192], s4c)
                  + _dot(u3w[:, 192:], jnp.concatenate(r2, axis=1)))


def _img_b(f8_ref, r_ref,
           u3W0, u3G0, u3B0, u3W1, u3G1, u3B1,
           out_ref, y_scr, z_scr, m2_scr, v2_scr):
    s = pl.program_id(0)
    u3w = u3W0[...]

    for k in range(4):
        @pl.when(s == k)
        def _build(k=k):
            dh3, dw3 = k // 2, k % 2
            t = _dot(u3w[:, :128], f8_ref[...])
            r = r_ref[...]
            wsel = (jax.lax.broadcasted_iota(jnp.int32, (128, 64), 0)
                    == 2 * jax.lax.broadcasted_iota(jnp.int32, (128, 64), 1)
                    + dw3).astype(jnp.float32)
            hsel = (2 * jax.lax.broadcasted_iota(jnp.int32, (20, 40), 0)
                    + dh3
                    == jax.lax.broadcasted_iota(jnp.int32, (20, 40), 1)
                    ).astype(jnp.float32)
            parts = []
            for b in range(2):
                rb = r[:, b * 5120:(b + 1) * 5120].reshape(64, 40, 128)
                rh = jax.lax.dot_general(
                    hsel, rb, (((1,), (1,)), ((), ())),
                    preferred_element_type=jnp.float32)
                rh = jnp.transpose(rh, (1, 0, 2))
                rv = jax.lax.dot_general(
                    rh, wsel, (((2,), (0,)), ((), ())),
                    preferred_element_type=jnp.float32)
                parts.append(rv.reshape(64, 1280))
            y_scr[k] = t + jnp.concatenate(parts, axis=1)

    @pl.when(s == 4)
    def _stats():
        y_all = y_scr[...]
        m = jnp.mean(y_all, axis=(0, 2), keepdims=True)
        v = jnp.mean((y_all - m) ** 2, axis=(0, 2), keepdims=True)
        g0 = u3G0[...]
        b0 = u3B0[...]
        for q in range(4):
            hq = jax.nn.relu(g0 * (y_scr[q] - m[0]) / jnp.sqrt(v[0] + _EPS)
                             + b0)
            z_scr[q] = _dot(u3W1[...], hq)
        z_all = z_scr[...]
        m2 = jnp.mean(z_all, axis=(0, 2), keepdims=True)
        v2 = jnp.mean((z_all - m2) ** 2, axis=(0, 2), keepdims=True)
        m2_scr[...] = m2[0]
        v2_scr[...] = v2[0]

    for k in range(4, 8):
        @pl.when(s == k)
        def _write(k=k):
            z = z_scr[k - 4]
            o = jax.nn.relu(u3G1[...] * (z - m2_scr[...])
                            / jnp.sqrt(v2_scr[...] + _EPS) + u3B1[...])
            o4 = jnp.concatenate([o[:, :1280].reshape(1, 64, 20, 64),
                                  o[:, 1280:].reshape(1, 64, 20, 64)],
                                 axis=0)
            out_ref[...] = o4.reshape(1, 1, 2, 64, 20, 64)


def _cols(x):
    return x.transpose(1, 0, 2).reshape(x.shape[1], -1)


def _layers(p):
    out = []
    for w, g, b in p:
        out.extend([w, g.reshape(-1, 1), b.reshape(-1, 1)])
    return out


def _layers_nolast(p):
    out = []
    for w, g, b in p[:-1]:
        out.extend([w, g.reshape(-1, 1), b.reshape(-1, 1)])
    out.append(p[-1][0])
    return out


def kernel(pc, node_a, node_b, img_s32_feature_map, img_s16_feature_map,
           img_s8_feature_map, img_s4_feature_map, img_s2_feature_map,
           img_global_feature, global_feature, node_b_features,
           node_a_features, node_a_min_k_idx, params):
    f32 = jnp.float32
    n = pc.shape[2]
    nbf_c = _cols(node_b_features)
    naf_c = _cols(node_a_features)
    ig_c = jnp.broadcast_to(img_global_feature.T[:, :, None],
                            (512, 2, 128)).reshape(512, 256)
    gf_c = jnp.broadcast_to(global_feature.transpose(1, 0, 2),
                            (512, 2, 128)).reshape(512, 256)
    s32f = img_s32_feature_map.reshape(2, 512, 80)
    s16f = img_s16_feature_map.reshape(2, 256, 320)
    p = params
    up_nb_c, up_na_c = pl.pallas_call(
        _point_kernel,
        out_shape=[jax.ShapeDtypeStruct((512, 256), f32),
                   jax.ShapeDtypeStruct((128, 256), f32)],
    )(nbf_c, naf_c, ig_c, gf_c, s32f, s16f, node_a, node_b,
      *_layers_nolast(p["nb_att"]), *_layers_nolast(p["nb_pn"]),
      *_layers_nolast(p["na_att"]), *_layers_nolast(p["na_pn"]))

    up_nb3 = up_nb_c.reshape(512, 2, 128).transpose(1, 0, 2)
    up_na3 = up_na_c.reshape(128, 2, 128).transpose(1, 0, 2)
    idx_t = node_a_min_k_idx.astype(jnp.int32).transpose(0, 2, 1)
    interp_pb, interp_pa = pl.pallas_call(
        _knn_kernel,
        grid=(2, n // _CHUNK),
        compiler_params=pltpu.CompilerParams(
            dimension_semantics=("parallel", "parallel")),
        in_specs=[
            pl.BlockSpec((1, 3, _CHUNK), lambda b, i: (b, 0, i)),
            pl.BlockSpec((1, 3, 128), lambda b, i: (b, 0, 0)),
            pl.BlockSpec((1, 3, 128), lambda b, i: (b, 0, 0)),
            pl.BlockSpec((1, 512, 128), lambda b, i: (b, 0, 0)),
            pl.BlockSpec((1, 128, 128), lambda b, i: (b, 0, 0)),
            pl.BlockSpec((1, 3, _CHUNK), lambda b, i: (b, 0, i)),
        ],
        out_specs=[
            pl.BlockSpec((1, 512, _CHUNK), lambda b, i: (b, 0, i)),
            pl.BlockSpec((1, 128, _CHUNK), lambda b, i: (b, 0, i)),
        ],
        out_shape=[jax.ShapeDtypeStruct((2, 512, n), f32),
                   jax.ShapeDtypeStruct((2, 128, n), f32)],
    )(pc, node_a, node_b, up_nb3, up_na3, idx_t)

    s32c = _cols(s32f)
    s16c = _cols(img_s16_feature_map.reshape(2, 256, 320))
    s8c = _cols(img_s8_feature_map.reshape(2, 128, 1280))
    g32c = jnp.broadcast_to(global_feature.transpose(1, 0, 2),
                            (512, 2, 80)).reshape(512, 160)
    g16c = jnp.broadcast_to(global_feature.transpose(1, 0, 2),
                            (512, 2, 320)).reshape(512, 640)
    f8c, rc = pl.pallas_call(
        _img_a,
        out_shape=[jax.ShapeDtypeStruct((128, 2560), f32),
                   jax.ShapeDtypeStruct((64, 10240), f32)],
    )(s32c, s16c, s8c, img_s4_feature_map, img_s2_feature_map, g32c, g16c,
      node_b_features, node_a_features,
      *_layers_nolast(p["att32"]), *_layers_nolast(p["att16"]),
      *_layers(p["up1"]), *_layers(p["up2"]), p["up3"][0][0])

    up3l = _layers(p["up3"])
    fmid_r = pl.pallas_call(
        _img_b,
        grid=(8,),
        in_specs=[
            pl.BlockSpec((128, 2560), lambda s: (0, 0)),
            pl.BlockSpec((64, 10240), lambda s: (0, 0)),
        ] + [pl.BlockSpec(w.shape, lambda s: (0, 0)) for w in up3l],
        out_specs=pl.BlockSpec(
            (1, 1, 2, 64, 20, 64),
            lambda s: (jnp.maximum(s - 4, 0) // 2, jnp.maximum(s - 4, 0) % 2,
                       0, 0, 0, 0)),
        out_shape=jax.ShapeDtypeStruct((2, 2, 2, 64, 20, 64), f32),
        scratch_shapes=[pltpu.VMEM((4, 64, 2560), f32),
                        pltpu.VMEM((4, 64, 2560), f32),
                        pltpu.VMEM((64, 1), f32),
                        pltpu.VMEM((64, 1), f32)],
    )(f8c, rc, *up3l)
    fmid = (fmid_r.transpose(2, 3, 4, 0, 5, 1).reshape(2, 64, 40, 128))
    return (fmid, interp_pa, interp_pb)


# speedup vs baseline: 1.0869x; 1.0117x over previous
import jax
import jax.numpy as jnp
from jax.experimental import pallas as pl
from jax.experimental.pallas import tpu as pltpu

_EPS = 1e-5
_CHUNK = 4096


def _dot(a, b):
    return jax.lax.dot_general(a, b, (((1,), (0,)), ((), ())),
                               preferred_element_type=jnp.float32)


def _dot_t(a, b):
    return jax.lax.dot_general(a, b, (((1,), (1,)), ((), ())),
                               preferred_element_type=jnp.float32)


def _bn_act(y, g, b):
    m = jnp.mean(y, axis=1, keepdims=True)
    v = jnp.mean((y - m) ** 2, axis=1, keepdims=True)
    return jax.nn.relu(g * (y - m) / jnp.sqrt(v + _EPS) + b)


def _softmax_rows(y):
    z = y - jnp.max(y, axis=0, keepdims=True)
    e = jnp.exp(z)
    return e / jnp.sum(e, axis=0, keepdims=True)


def _top3_weights(d, iota):
    dw = d
    cnt = jnp.zeros_like(d)
    s = jnp.zeros(d.shape[:1] + (1,), d.dtype)
    for _ in range(3):
        m = jnp.min(dw, axis=1, keepdims=True)
        eq = dw == m
        ji = jnp.min(jnp.where(eq, iota, jnp.int32(1 << 30)), axis=1,
                     keepdims=True)
        E = iota == ji
        cnt = cnt + E.astype(jnp.float32)
        s = s + m
        dw = jnp.where(E, jnp.float32(3e38), dw)
    return (1.0 - d / s) * cnt


def _pair_d(a, b):
    d2 = ((a[0][:, None] - b[0][None, :]) ** 2
          + (a[1][:, None] - b[1][None, :]) ** 2
          + (a[2][:, None] - b[2][None, :]) ** 2)
    return jnp.sqrt(d2)


def _up_mat(hw_lo, w_hi):
    hw_hi = 4 * hw_lo
    ri = jax.lax.broadcasted_iota(jnp.int32, (hw_lo, hw_hi), 0)
    cj = jax.lax.broadcasted_iota(jnp.int32, (hw_lo, hw_hi), 1)
    parent = (cj // (2 * w_hi)) * (w_hi // 2) + (cj % w_hi) // 2
    return (ri == parent).astype(jnp.float32)


def _up2x_nat(t, u):
    c = t.shape[0]
    hw = t.shape[1] // 2
    tt = jnp.concatenate([t[:, :hw], t[:, hw:]], axis=0)
    up = _dot(tt, u)
    return jnp.concatenate([up[:c], up[c:]], axis=1)


def _point_kernel(nbf_ref, naf_ref, ig_ref, gf_ref, s32f_ref, s16f_ref,
                  na_ref, nb_ref,
                  baW0, baG0, baB0, baW1,
                  bpW0, bpG0, bpB0, bpW1, bpG1, bpB1, bpW2,
                  aaW0, aaG0, aaB0, aaW1,
                  apW0, apG0, apB0, apW1, apG1, apB1, apW2,
                  up_nb_out, up_na_out):
    nbf = nbf_ref[...]
    naf = naf_ref[...]
    ig = ig_ref[...]
    gf = gf_ref[...]
    x = jnp.concatenate([nbf, ig], axis=0)
    h = _bn_act(_dot(baW0[...], x), baG0[...], baB0[...])
    att = _softmax_rows(_dot(baW1[...], h))
    w32 = jnp.concatenate(
        [_dot(s32f_ref[b], att[:, b * 128:(b + 1) * 128]) for b in range(2)],
        axis=1)
    x2 = jnp.concatenate([nbf, gf, w32, ig], axis=0)
    h = _bn_act(_dot(bpW0[...], x2), bpG0[...], bpB0[...])
    h = _bn_act(_dot(bpW1[...], h), bpG1[...], bpB1[...])
    up_nb = _dot(bpW2[...], h)
    up_nb_out[...] = up_nb
    x3 = jnp.concatenate([naf, ig], axis=0)
    h = _bn_act(_dot(aaW0[...], x3), aaG0[...], aaB0[...])
    att16 = _softmax_rows(_dot(aaW1[...], h))
    w16 = jnp.concatenate(
        [_dot(s16f_ref[b], att16[:, b * 128:(b + 1) * 128]) for b in range(2)],
        axis=1)
    iota = jax.lax.broadcasted_iota(jnp.int32, (128, 128), 1)
    interp_ab = jnp.concatenate(
        [_dot_t(up_nb[:, b * 128:(b + 1) * 128],
                _top3_weights(_pair_d(na_ref[b], nb_ref[b]), iota))
         for b in range(2)], axis=1)
    x4 = jnp.concatenate([naf, interp_ab, w16], axis=0)
    h = _bn_act(_dot(apW0[...], x4), apG0[...], apB0[...])
    h = _bn_act(_dot(apW1[...], h), apG1[...], apB1[...])
    up_na_out[...] = _dot(apW2[...], h)


def _knn_kernel(pc_ref, na_ref, nb_ref, upnb_ref, upna_ref, idx_ref,
                pb_out, pa_out):
    pc = pc_ref[0]
    c = pc.shape[1]
    iota = jax.lax.broadcasted_iota(jnp.int32, (c, 128), 1)
    db = _pair_d(pc, nb_ref[0])
    wb = _top3_weights(db, iota)
    pb_out[0] = _dot_t(upnb_ref[0], wb)
    da = _pair_d(pc, na_ref[0])
    idx = idx_ref[0]
    cnt = ((idx[0][:, None] == iota).astype(jnp.float32)
           + (idx[1][:, None] == iota).astype(jnp.float32)
           + (idx[2][:, None] == iota).astype(jnp.float32))
    s = jnp.sum(cnt * da, axis=1, keepdims=True)
    wa = (1.0 - da / s) * cnt
    pa_out[0] = _dot_t(upna_ref[0], wa)


def _img_a(s32_ref, s16_ref, s8_ref, s4_ref, s2_ref, g32_ref, g16_ref,
           nbf_ref, naf_ref,
           a32W0, a32G0, a32B0, a32W1, a32G1, a32B1, a32W2,
           a16W0, a16G0, a16B0, a16W1, a16G1, a16B1, a16W2,
           u1W0, u1G0, u1B0, u1W1, u1G1, u1B1,
           u2W0, u2G0, u2B0, u2W1, u2G1, u2B1,
           u3W0,
           f8_out, r_out):
    s32 = s32_ref[...]
    s16 = jnp.concatenate([s16_ref[0].reshape(256, 320),
                           s16_ref[1].reshape(256, 320)], axis=1)
    s8 = jnp.concatenate([s8_ref[0].reshape(128, 1280),
                          s8_ref[1].reshape(128, 1280)], axis=1)
    x = jnp.concatenate([s32, g32_ref[...]], axis=0)
    h = _bn_act(_dot(a32W0[...], x), a32G0[...], a32B0[...])
    h = _bn_act(_dot(a32W1[...], h), a32G1[...], a32B1[...])
    a32 = _softmax_rows(_dot(a32W2[...], h))
    fus32 = jnp.concatenate(
        [jnp.concatenate(
            [_dot(nbf_ref[b], a32[:, b * 80:(b + 1) * 80]) for b in range(2)],
            axis=1), s32], axis=0)
    x = jnp.concatenate([s16, g16_ref[...]], axis=0)
    h = _bn_act(_dot(a16W0[...], x), a16G0[...], a16B0[...])
    h = _bn_act(_dot(a16W1[...], h), a16G1[...], a16B1[...])
    a16 = _softmax_rows(_dot(a16W2[...], h))
    fus16 = jnp.concatenate(
        [jnp.concatenate(
            [_dot(naf_ref[b], a16[:, b * 320:(b + 1) * 320]) for b in range(2)],
            axis=1), s16], axis=0)
    u1w = u1W0[...]
    y = (_up2x_nat(_dot(u1w[:, :768], fus32), _up_mat(80, 32))
         + _dot(u1w[:, 768:], fus16))
    h = _bn_act(y, u1G0[...], u1B0[...])
    f16 = _bn_act(_dot(u1W1[...], h), u1G1[...], u1B1[...])
    u2w = u2W0[...]
    y = (_up2x_nat(_dot(u2w[:, :256], f16), _up_mat(320, 64))
         + _dot(u2w[:, 256:], s8))
    h = _bn_act(y, u2G0[...], u2B0[...])
    f8_out[...] = _bn_act(_dot(u2W1[...], h), u2G1[...], u2B1[...])
    u3w = u3W0[...]
    hsel = (2 * jax.lax.broadcasted_iota(jnp.int32, (40, 80), 0)
            == jax.lax.broadcasted_iota(jnp.int32, (40, 80), 1)
            ).astype(jnp.float32)
    wsel = (jax.lax.broadcasted_iota(jnp.int32, (256, 128), 0)
            == 2 * jax.lax.broadcasted_iota(jnp.int32, (256, 128), 1)
            ).astype(jnp.float32)
    r2 = []
    for b in range(2):
        rh = jax.lax.dot_general(hsel, s2_ref[b],
                                 (((1,), (1,)), ((), ())),
                                 preferred_element_type=jnp.float32)
        rh = jnp.transpose(rh, (1, 0, 2))
        qw = jax.lax.dot_general(rh, wsel, (((2,), (0,)), ((), ())),
                                 preferred_element_type=jnp.float32)
        r2.append(qw.reshape(64, 5120))
    s4c = jnp.concatenate([s4_ref[0].reshape(64, 5120),
                           s4_ref[1].reshape(64, 5120)], axis=1)
    r_out[...] = (_dot(u3w[:, 128:192], s4c)
                  + _dot(u3w[:, 192:], jnp.concatenate(r2, axis=1)))


def _img_b(f8_ref, r_ref,
           u3W0, u3G0, u3B0, u3W1, u3G1, u3B1,
           out_ref, y_scr, z_scr, m2_scr, v2_scr):
    s = pl.program_id(0)
    u3w = u3W0[...]

    for k in range(4):
        @pl.when(s == k)
        def _build(k=k):
            dh3, dw3 = k // 2, k % 2
            t = _dot(u3w[:, :128], f8_ref[...])
            r = r_ref[...]
            wsel = (jax.lax.broadcasted_iota(jnp.int32, (128, 64), 0)
                    == 2 * jax.lax.broadcasted_iota(jnp.int32, (128, 64), 1)
                    + dw3).astype(jnp.float32)
            hsel = (2 * jax.lax.broadcasted_iota(jnp.int32, (20, 40), 0)
                    + dh3
                    == jax.lax.broadcasted_iota(jnp.int32, (20, 40), 1)
                    ).astype(jnp.float32)
            parts = []
            for b in range(2):
                rb = r[:, b * 5120:(b + 1) * 5120].reshape(64, 40, 128)
                rh = jax.lax.dot_general(
                    hsel, rb, (((1,), (1,)), ((), ())),
                    preferred_element_type=jnp.float32)
                rh = jnp.transpose(rh, (1, 0, 2))
                rv = jax.lax.dot_general(
                    rh, wsel, (((2,), (0,)), ((), ())),
                    preferred_element_type=jnp.float32)
                parts.append(rv.reshape(64, 1280))
            y_scr[k] = t + jnp.concatenate(parts, axis=1)

    @pl.when(s == 4)
    def _stats():
        y_all = y_scr[...]
        m = jnp.mean(y_all, axis=(0, 2), keepdims=True)
        v = jnp.mean((y_all - m) ** 2, axis=(0, 2), keepdims=True)
        g0 = u3G0[...]
        b0 = u3B0[...]
        for q in range(4):
            hq = jax.nn.relu(g0 * (y_scr[q] - m[0]) / jnp.sqrt(v[0] + _EPS)
                             + b0)
            z_scr[q] = _dot(u3W1[...], hq)
        z_all = z_scr[...]
        m2 = jnp.mean(z_all, axis=(0, 2), keepdims=True)
        v2 = jnp.mean((z_all - m2) ** 2, axis=(0, 2), keepdims=True)
        m2_scr[...] = m2[0]
        v2_scr[...] = v2[0]

    for k in range(4, 8):
        @pl.when(s == k)
        def _write(k=k):
            z = z_scr[k - 4]
            o = jax.nn.relu(u3G1[...] * (z - m2_scr[...])
                            / jnp.sqrt(v2_scr[...] + _EPS) + u3B1[...])
            o4 = jnp.concatenate([o[:, :1280].reshape(1, 64, 20, 64),
                                  o[:, 1280:].reshape(1, 64, 20, 64)],
                                 axis=0)
            out_ref[...] = o4.reshape(1, 1, 2, 64, 20, 64)


def _cols(x):
    return x.transpose(1, 0, 2).reshape(x.shape[1], -1)


def _layers(p):
    out = []
    for w, g, b in p:
        out.extend([w, g.reshape(-1, 1), b.reshape(-1, 1)])
    return out


def _layers_nolast(p):
    out = []
    for w, g, b in p[:-1]:
        out.extend([w, g.reshape(-1, 1), b.reshape(-1, 1)])
    out.append(p[-1][0])
    return out


def kernel(pc, node_a, node_b, img_s32_feature_map, img_s16_feature_map,
           img_s8_feature_map, img_s4_feature_map, img_s2_feature_map,
           img_global_feature, global_feature, node_b_features,
           node_a_features, node_a_min_k_idx, params):
    f32 = jnp.float32
    n = pc.shape[2]
    nbf_c = _cols(node_b_features)
    naf_c = _cols(node_a_features)
    ig_c = jnp.broadcast_to(img_global_feature.T[:, :, None],
                            (512, 2, 128)).reshape(512, 256)
    gf_c = jnp.broadcast_to(global_feature.transpose(1, 0, 2),
                            (512, 2, 128)).reshape(512, 256)
    s32f = img_s32_feature_map.reshape(2, 512, 80)
    s16f = img_s16_feature_map.reshape(2, 256, 320)
    p = params
    up_nb_c, up_na_c = pl.pallas_call(
        _point_kernel,
        out_shape=[jax.ShapeDtypeStruct((512, 256), f32),
                   jax.ShapeDtypeStruct((128, 256), f32)],
    )(nbf_c, naf_c, ig_c, gf_c, s32f, s16f, node_a, node_b,
      *_layers_nolast(p["nb_att"]), *_layers_nolast(p["nb_pn"]),
      *_layers_nolast(p["na_att"]), *_layers_nolast(p["na_pn"]))

    up_nb3 = up_nb_c.reshape(512, 2, 128).transpose(1, 0, 2)
    up_na3 = up_na_c.reshape(128, 2, 128).transpose(1, 0, 2)
    idx_t = node_a_min_k_idx.astype(jnp.int32).transpose(0, 2, 1)
    interp_pb, interp_pa = pl.pallas_call(
        _knn_kernel,
        grid=(2, n // _CHUNK),
        compiler_params=pltpu.CompilerParams(
            dimension_semantics=("parallel", "parallel")),
        in_specs=[
            pl.BlockSpec((1, 3, _CHUNK), lambda b, i: (b, 0, i)),
            pl.BlockSpec((1, 3, 128), lambda b, i: (b, 0, 0)),
            pl.BlockSpec((1, 3, 128), lambda b, i: (b, 0, 0)),
            pl.BlockSpec((1, 512, 128), lambda b, i: (b, 0, 0)),
            pl.BlockSpec((1, 128, 128), lambda b, i: (b, 0, 0)),
            pl.BlockSpec((1, 3, _CHUNK), lambda b, i: (b, 0, i)),
        ],
        out_specs=[
            pl.BlockSpec((1, 512, _CHUNK), lambda b, i: (b, 0, i)),
            pl.BlockSpec((1, 128, _CHUNK), lambda b, i: (b, 0, i)),
        ],
        out_shape=[jax.ShapeDtypeStruct((2, 512, n), f32),
                   jax.ShapeDtypeStruct((2, 128, n), f32)],
    )(pc, node_a, node_b, up_nb3, up_na3, idx_t)

    s32c = _cols(s32f)
    g32c = jnp.broadcast_to(global_feature.transpose(1, 0, 2),
                            (512, 2, 80)).reshape(512, 160)
    g16c = jnp.broadcast_to(global_feature.transpose(1, 0, 2),
                            (512, 2, 320)).reshape(512, 640)
    f8c, rc = pl.pallas_call(
        _img_a,
        out_shape=[jax.ShapeDtypeStruct((128, 2560), f32),
                   jax.ShapeDtypeStruct((64, 10240), f32)],
    )(s32c, img_s16_feature_map.reshape(2, 256, 320),
      img_s8_feature_map.reshape(2, 128, 1280),
      img_s4_feature_map, img_s2_feature_map, g32c, g16c,
      node_b_features, node_a_features,
      *_layers_nolast(p["att32"]), *_layers_nolast(p["att16"]),
      *_layers(p["up1"]), *_layers(p["up2"]), p["up3"][0][0])

    up3l = _layers(p["up3"])
    fmid_r = pl.pallas_call(
        _img_b,
        grid=(8,),
        in_specs=[
            pl.BlockSpec((128, 2560), lambda s: (0, 0)),
            pl.BlockSpec((64, 10240), lambda s: (0, 0)),
        ] + [pl.BlockSpec(w.shape, lambda s: (0, 0)) for w in up3l],
        out_specs=pl.BlockSpec(
            (1, 1, 2, 64, 20, 64),
            lambda s: (jnp.maximum(s - 4, 0) // 2, jnp.maximum(s - 4, 0) % 2,
                       0, 0, 0, 0)),
        out_shape=jax.ShapeDtypeStruct((2, 2, 2, 64, 20, 64), f32),
        scratch_shapes=[pltpu.VMEM((4, 64, 2560), f32),
                        pltpu.VMEM((4, 64, 2560), f32),
                        pltpu.VMEM((64, 1), f32),
                        pltpu.VMEM((64, 1), f32)],
    )(f8c, rc, *up3l)
    fmid = (fmid_r.transpose(2, 3, 4, 0, 5, 1).reshape(2, 64, 40, 128))
    return (fmid, interp_pa, interp_pb)
